# Initial kernel scaffold; baseline (speedup 1.0000x reference)
import jax
import jax.numpy as jnp
from jax.experimental import pallas as pl


def kernel(x, edge_index, edge_weight, W1, b1, W2, b2):
    raise NotImplementedError("write your pallas kernel here")


# speedup vs baseline: 6.0125x; 6.0125x over previous
import functools

import jax
import jax.numpy as jnp
from jax import lax
from jax.experimental import pallas as pl
from jax.experimental.pallas import tpu as pltpu
from jax.experimental.pallas import tpu_sc as plsc

NC = 2
NS = 16
NW = NC * NS
CH = 128


def _make_spmm(n, d, nch):
    rpt = -(-(n // NS) // 8) * 8
    rlast = n - (NS - 1) * rpt
    assert rlast > 0 and rlast % 8 == 0
    nreg = d // 16
    mesh = plsc.VectorSubcoreMesh(core_axis_name="c", subcore_axis_name="s",
                                  num_cores=NC)

    @functools.partial(
        pl.kernel,
        mesh=mesh,
        compiler_params=pltpu.CompilerParams(
            use_tc_tiling_on_sc=(d % 128 == 0)),
        out_type=jax.ShapeDtypeStruct((NC, n, d), jnp.float32),
        scratch_types=[
            pltpu.VMEM((nch, CH), jnp.int32),
            pltpu.VMEM((nch, CH), jnp.int32),
            pltpu.VMEM((nch * CH,), jnp.float32),
            pltpu.VMEM((CH, d), jnp.float32),
            pltpu.VMEM_SHARED((n, d), jnp.float32),
            pltpu.SemaphoreType.DMA,
        ],
    )
    def spmm(feat, srcw, dstw, ww, zeros, out, src_v, dst_v, w_v, rows_v,
             acc, sem):
        cid = lax.axis_index("c")
        sid = lax.axis_index("s")
        wid = cid * NS + sid
        base = pl.multiple_of(sid * rpt, 8)

        @pl.when(sid < NS - 1)
        def _():
            pltpu.sync_copy(zeros.at[pl.ds(base, rpt)],
                            acc.at[pl.ds(base, rpt)])

        @pl.when(sid == NS - 1)
        def _():
            pltpu.sync_copy(zeros.at[pl.ds((NS - 1) * rpt, rlast)],
                            acc.at[pl.ds((NS - 1) * rpt, rlast)])

        pltpu.sync_copy(srcw.at[wid], src_v)
        pltpu.sync_copy(dstw.at[wid], dst_v)
        pltpu.sync_copy(ww.at[wid], w_v)
        plsc.subcore_barrier()

        def chunk_body(j, carry):
            pltpu.async_copy(feat.at[src_v.at[j]], rows_v, sem).wait()

            def grp_body(eg, c2):
                wrow = w_v[pl.ds(pl.multiple_of(j * CH + eg * 16, 16), 16)]
                for el in range(16):
                    lane = jnp.full((16,), el, jnp.int32)
                    wv = wrow.at[lane].get(mode="promise_in_bounds")
                    e = eg * 16 + el
                    for k in range(nreg):
                        sl = pl.ds(k * 16, 16)
                        rows_v[e, sl] = rows_v[e, sl] * wv
                return c2

            lax.fori_loop(0, CH // 16, grp_body, 0)
            pltpu.sync_copy(rows_v, acc.at[dst_v.at[j]], add=True)
            return carry

        lax.fori_loop(0, nch, chunk_body, 0)
        plsc.subcore_barrier()

        @pl.when(sid < NS - 1)
        def _():
            pltpu.sync_copy(acc.at[pl.ds(base, rpt)],
                            out.at[cid, pl.ds(base, rpt)])

        @pl.when(sid == NS - 1)
        def _():
            pltpu.sync_copy(acc.at[pl.ds((NS - 1) * rpt, rlast)],
                            out.at[cid, pl.ds((NS - 1) * rpt, rlast)])

    return spmm


def _mm1_body(x_ref, w_ref, b_ref, o_ref):
    o_ref[...] = (jnp.dot(x_ref[...], w_ref[...],
                          preferred_element_type=jnp.float32) + b_ref[...])


def _mm2_body(p_ref, q_ref, w_ref, b_ref, o_ref):
    h = p_ref[...] + q_ref[0] + q_ref[1]
    h = jnp.maximum(h, 0.0)
    o_ref[...] = (jnp.dot(h, w_ref[...],
                          preferred_element_type=jnp.float32) + b_ref[...])


def _make_out_body(nclass):
    def body(g_ref, r_ref, o_ref):
        z = g_ref[...] + r_ref[0] + r_ref[1]
        col = lax.broadcasted_iota(jnp.int32, z.shape, 1)
        z = jnp.where(col < nclass, z, -1e30)
        m = jnp.max(z, axis=1, keepdims=True)
        s = jnp.sum(jnp.exp(z - m), axis=1, keepdims=True)
        o_ref[...] = (z - m - jnp.log(s))[:, :nclass]
    return body


def kernel(x, edge_index, edge_weight, W1, b1, W2, b2):
    n, f = x.shape
    e = edge_weight.shape[0]
    nhid = W1.shape[1]
    nclass = W2.shape[1]
    dpad = ((nclass + 15) // 16) * 16

    nch = -(-e // (NW * CH))
    ep = NW * nch * CH
    pad = ep - e

    src = jnp.concatenate([edge_index[0], jnp.zeros((pad,), jnp.int32)])
    dst = jnp.concatenate([edge_index[1], jnp.zeros((pad,), jnp.int32)])
    ew = jnp.concatenate([edge_weight, jnp.zeros((pad,), jnp.float32)])
    src = src.reshape(NW, nch, CH)
    dst = dst.reshape(NW, nch, CH)
    ew = ew.reshape(NW, nch * CH)
    zeros_h = jnp.zeros((n, nhid), jnp.float32)
    zeros_c = jnp.zeros((n, dpad), jnp.float32)

    W2p = jnp.pad(W2, ((0, 0), (0, dpad - nclass)))
    b1r = b1.reshape(1, nhid)
    b2r = jnp.pad(b2, (0, dpad - nclass)).reshape(1, dpad)

    rows = 1000
    grid = (n // rows,)

    p = pl.pallas_call(
        _mm1_body,
        grid=grid,
        in_specs=[
            pl.BlockSpec((rows, f), lambda i: (i, 0)),
            pl.BlockSpec((f, nhid), lambda i: (0, 0)),
            pl.BlockSpec((1, nhid), lambda i: (0, 0)),
        ],
        out_specs=pl.BlockSpec((rows, nhid), lambda i: (i, 0)),
        out_shape=jax.ShapeDtypeStruct((n, nhid), jnp.float32),
    )(x, W1, b1r)

    q = _make_spmm(n, nhid, nch)(p, src, dst, ew, zeros_h)

    g = pl.pallas_call(
        _mm2_body,
        grid=grid,
        in_specs=[
            pl.BlockSpec((rows, nhid), lambda i: (i, 0)),
            pl.BlockSpec((NC, rows, nhid), lambda i: (0, i, 0)),
            pl.BlockSpec((nhid, dpad), lambda i: (0, 0)),
            pl.BlockSpec((1, dpad), lambda i: (0, 0)),
        ],
        out_specs=pl.BlockSpec((rows, dpad), lambda i: (i, 0)),
        out_shape=jax.ShapeDtypeStruct((n, dpad), jnp.float32),
    )(p, q, W2p, b2r)

    r = _make_spmm(n, dpad, nch)(g, src, dst, ew, zeros_c)

    out = pl.pallas_call(
        _make_out_body(nclass),
        grid=grid,
        in_specs=[
            pl.BlockSpec((rows, dpad), lambda i: (i, 0)),
            pl.BlockSpec((NC, rows, dpad), lambda i: (0, i, 0)),
        ],
        out_specs=pl.BlockSpec((rows, nclass), lambda i: (i, 0)),
        out_shape=jax.ShapeDtypeStruct((n, nclass), jnp.float32),
    )(g, r)
    return out
